# baseline (speedup 1.0000x reference)
import functools

import jax
import jax.numpy as jnp
from jax.experimental import pallas as pl
from jax.experimental.pallas import tpu as pltpu


def _fused_cell(x_ref, wz_ref, bz_ref, wh_ref, bh_ref, wlin_ref, blin_ref,
                o_ref, *, out_ch):
    xb = x_ref[...]
    s = jnp.tanh(
        jnp.dot(xb, wz_ref[...], preferred_element_type=jnp.float32)
        + bz_ref[...])
    t = jnp.tanh(
        jnp.dot(xb, wh_ref[...], preferred_element_type=jnp.float32)
        + bh_ref[...])
    h = jnp.maximum((1.0 - s) * t, 0.0)
    r = jax.lax.dot_general(wlin_ref[...], h, (((1,), (1,)), ((), ())),
                            preferred_element_type=jnp.float32)
    o_ref[...] = r[0] + blin_ref[0, 0]


def kernel(x, edge_index, edge_weight, W_z, b_z, W_r, b_r, W_h, b_h,
           W_lin, b_lin):
    del edge_index, edge_weight, W_r, b_r
    n, in_ch = x.shape
    out_ch = W_z.shape[-1]

    wz = 0.5 * (W_z[0, 0, :in_ch, :] + W_z[1, 0, :in_ch, :])
    wh = W_h[0, 0, :in_ch, :] + W_h[1, 0, :in_ch, :]
    bz = 0.5 * b_z.reshape(1, out_ch)
    bh = b_h.reshape(1, out_ch)
    wlin = 0.5 * W_lin.reshape(1, out_ch)
    blin = b_lin.reshape(1, 1)

    block = 5120
    grid = (n + block - 1) // block

    full = lambda i: (0, 0)
    out1d = pl.pallas_call(
        functools.partial(_fused_cell, out_ch=out_ch),
        grid=(grid,),
        in_specs=[
            pl.BlockSpec((block, in_ch), lambda i: (i, 0)),
            pl.BlockSpec((in_ch, out_ch), full),
            pl.BlockSpec((1, out_ch), full),
            pl.BlockSpec((in_ch, out_ch), full),
            pl.BlockSpec((1, out_ch), full),
            pl.BlockSpec((1, out_ch), full),
            pl.BlockSpec((1, 1), full),
        ],
        out_specs=pl.BlockSpec((block,), lambda i: (i,)),
        out_shape=jax.ShapeDtypeStruct((grid * block,), x.dtype),
        compiler_params=pltpu.CompilerParams(
            dimension_semantics=("parallel",)),
    )(x, wz, bz, wh, bh, wlin, blin)
    return out1d[:n, None]
